# Initial kernel scaffold; baseline (speedup 1.0000x reference)
import functools

import jax
import jax.numpy as jnp
import numpy as np
from jax.experimental import pallas as pl
from jax.experimental.pallas import tpu as pltpu

_ADJ = (np.array(
    [[1, 0, 0, 0, 1, 0], [1, 1, 0, 0, 0, 1], [1, 0, 1, 0, 0, 1],
     [1, 0, 0, 1, 0, 0], [1, 0, 0, 0, 1, 0], [0, 1, 1, 0, 0, 1]],
    dtype=np.int64).T != 0)

_SHIFTS = []
for _d in range(-5, 6):
    _rs = [r for r in range(6) if 0 <= r + _d < 6 and _ADJ[r, r + _d]]
    if _rs:
        _SHIFTS.append((_d, tuple(_rs)))

_NEG = -1e30
_SLOPE = 0.2


def _leaky(v):
    return jnp.where(v >= 0, v, _SLOPE * v)


def _matmul_t(a, w):
    return jax.lax.dot_general(a, w, (((1,), (1,)), ((), ())),
                               preferred_element_type=jnp.float32)


def _channel_onehot(row0, n_rows, period, group):
    rows = row0 + jax.lax.broadcasted_iota(jnp.int32, (n_rows, 1), 0)
    ch = jax.lax.rem(rows, period) // group
    lanes = jax.lax.broadcasted_iota(jnp.int32, (n_rows, 64), 1)
    return (ch == lanes).astype(jnp.float32)


def _pass1_body(nlayer, gb, tv, v_per_g, d_rows,
                xg_ref, xd_ref, xs_ref, wl_ref, bl_ref, wr_ref, br_ref,
                att_ref, b_ref, base_ref, a_ref, stats_ref):
    j = pl.program_id(0)
    xg = xg_ref[...]
    xd = xd_ref[...]
    xs = xs_ref[...]

    nr_b = xg.shape[0]
    nr_a = xd.shape[0]
    rmod = jax.lax.rem(jax.lax.broadcasted_iota(jnp.int32, (nr_a, 1), 0), 6)

    base_acc = jnp.zeros((nr_b, 64), jnp.float32)
    a_acc = jnp.zeros((nr_a, 64), jnp.float32)

    for i in range(nlayer):
        wl = wl_ref[i]
        wr = wr_ref[i]
        bl_i = bl_ref[i:i + 1, :]
        br_i = br_ref[i:i + 1, :]
        att_i = att_ref[i:i + 1, :]
        b_i = b_ref[i:i + 1, :]

        xl = _matmul_t(xg, wl) + bl_i
        s = _matmul_t(xs, wl) + bl_i
        xld = _matmul_t(xd, wl) + bl_i
        xrd = _matmul_t(xd, wr) + br_i

        sl = jnp.sum(_leaky(xld + xrd) * att_i, axis=1, keepdims=True)

        s_roll = []
        e_list = []
        mask_list = []
        for d, rset in _SHIFTS:
            sr = jnp.roll(s, -d, axis=0)
            s_roll.append(sr)
            e = jnp.sum(_leaky(sr + xrd) * att_i, axis=1, keepdims=True)
            if len(rset) == 6:
                mask = None
            else:
                mask = functools.reduce(
                    jnp.logical_or, [rmod == r for r in rset])
            e_list.append(e)
            mask_list.append(mask)

        m = sl
        for e, mask in zip(e_list, mask_list):
            em = e if mask is None else jnp.where(mask, e, _NEG)
            m = jnp.maximum(m, em)

        ex_self = jnp.exp(sl - m)
        den = ex_self
        mix = ex_self * xld
        for e, mask, sr in zip(e_list, mask_list, s_roll):
            ex = jnp.exp(e - m)
            if mask is not None:
                ex = jnp.where(mask, ex, 0.0)
            den = den + ex
            mix = mix + ex * sr

        a_acc = a_acc + mix / (den + 1e-16) + b_i
        base_acc = base_acc + xl + b_i

    base_ref[...] = base_acc
    a_ref[...] = a_acc

    row0_b = j * nr_b
    row0_a = j * nr_a
    rows_b = row0_b + jax.lax.broadcasted_iota(jnp.int32, (nr_b, 1), 0)
    tail = (rows_b >= d_rows).astype(jnp.float32)
    rs_b = jnp.sum(base_acc, axis=1, keepdims=True) * tail
    rss_b = jnp.sum(base_acc * base_acc, axis=1, keepdims=True) * tail
    rs_a = jnp.sum(a_acc, axis=1, keepdims=True)
    rss_a = jnp.sum(a_acc * a_acc, axis=1, keepdims=True)

    oh_b = _channel_onehot(row0_b, nr_b, tv, v_per_g)
    oh_a = _channel_onehot(row0_a, nr_a, tv, v_per_g)
    vals_b = jnp.concatenate([rs_b, rss_b], axis=1)
    vals_a = jnp.concatenate([rs_a, rss_a], axis=1)
    contrib = (
        jax.lax.dot_general(oh_b, vals_b, (((0,), (0,)), ((), ())),
                            preferred_element_type=jnp.float32)
        + jax.lax.dot_general(oh_a, vals_a, (((0,), (0,)), ((), ())),
                              preferred_element_type=jnp.float32))

    @pl.when(j == 0)
    def _():
        stats_ref[...] = contrib

    @pl.when(j != 0)
    def _():
        stats_ref[...] = stats_ref[...] + contrib


def _pass2_body(n_total, d_rows, tv, v_per_g,
                base_ref, a_ref, xres_ref, stats_ref, bnw_ref, bnb_ref,
                out_ref):
    j = pl.program_id(0)
    nr = out_ref.shape[0]
    cnt = float(n_total)

    s = stats_ref[:, 0:1]
    ss = stats_ref[:, 1:2]
    mean = s / cnt
    var = ss / cnt - mean * mean
    inv = jax.lax.rsqrt(var + 1e-5)
    scale_c = bnw_ref[...] * inv
    shift_c = bnb_ref[...] - mean * scale_c
    scsh = jnp.concatenate([scale_c, shift_c], axis=1)

    oh = _channel_onehot(j * nr, nr, tv, v_per_g)
    rowsc = jax.lax.dot_general(oh, scsh, (((1,), (0,)), ((), ())),
                                preferred_element_type=jnp.float32)
    scale_r = rowsc[:, 0:1]
    shift_r = rowsc[:, 1:2]

    rows = j * nr + jax.lax.broadcasted_iota(jnp.int32, (nr, 1), 0)
    y = jnp.where(rows < d_rows, a_ref[...], base_ref[...])
    out_ref[...] = jnp.maximum(y * scale_r + shift_r + xres_ref[...], 0.0)


def kernel(x, Wl, bl, Wr, br, att, b, bn_w, bn_b):
    NM, C, T, V = x.shape
    N = NM * T * V
    G = NM * T
    D = 6 * G
    TV = T * V

    resh = jnp.transpose(x, (0, 2, 3, 1)).reshape(N, C)
    srcflat = resh.reshape(G, V, C)[:, :6, :].reshape(6 * G, C)
    xflat = x.reshape(N, C)
    bnw2 = bn_w.reshape(C, 1)
    bnb2 = bn_b.reshape(C, 1)

    GB = 512
    grid1 = G // GB
    nb = V * GB
    na = 6 * GB

    p1 = pl.pallas_call(
        functools.partial(_pass1_body, 3, GB, TV, V, D),
        grid=(grid1,),
        in_specs=[
            pl.BlockSpec((nb, C), lambda j: (j, 0)),
            pl.BlockSpec((na, C), lambda j: (j, 0)),
            pl.BlockSpec((na, C), lambda j: (j, 0)),
            pl.BlockSpec((3, C, C), lambda j: (0, 0, 0)),
            pl.BlockSpec((3, C), lambda j: (0, 0)),
            pl.BlockSpec((3, C, C), lambda j: (0, 0, 0)),
            pl.BlockSpec((3, C), lambda j: (0, 0)),
            pl.BlockSpec((3, C), lambda j: (0, 0)),
            pl.BlockSpec((3, C), lambda j: (0, 0)),
        ],
        out_specs=[
            pl.BlockSpec((nb, C), lambda j: (j, 0)),
            pl.BlockSpec((na, C), lambda j: (j, 0)),
            pl.BlockSpec((C, 2), lambda j: (0, 0)),
        ],
        out_shape=[
            jax.ShapeDtypeStruct((N, C), jnp.float32),
            jax.ShapeDtypeStruct((D, C), jnp.float32),
            jax.ShapeDtypeStruct((C, 2), jnp.float32),
        ],
        compiler_params=pltpu.CompilerParams(
            dimension_semantics=("arbitrary",)),
    )
    base, a_out, stats = p1(resh, resh, srcflat, Wl, bl, Wr, br, att, b)

    R2 = 6144
    grid2 = N // R2
    head_blocks = D // R2

    p2 = pl.pallas_call(
        functools.partial(_pass2_body, N, D, TV, V),
        grid=(grid2,),
        in_specs=[
            pl.BlockSpec((R2, C), lambda j: (j, 0)),
            pl.BlockSpec((R2, C),
                         lambda j: (jnp.minimum(j, head_blocks - 1), 0)),
            pl.BlockSpec((R2, C), lambda j: (j, 0)),
            pl.BlockSpec((C, 2), lambda j: (0, 0)),
            pl.BlockSpec((C, 1), lambda j: (0, 0)),
            pl.BlockSpec((C, 1), lambda j: (0, 0)),
        ],
        out_specs=pl.BlockSpec((R2, C), lambda j: (j, 0)),
        out_shape=jax.ShapeDtypeStruct((N, C), jnp.float32),
        compiler_params=pltpu.CompilerParams(
            dimension_semantics=("arbitrary",)),
    )
    out = p2(base, a_out, xflat, stats, bnw2, bnb2)
    return out.reshape(NM, C, T, V)


# speedup vs baseline: 4.3808x; 4.3808x over previous
import functools

import jax
import jax.numpy as jnp
import numpy as np
from jax.experimental import pallas as pl
from jax.experimental.pallas import tpu as pltpu

_ADJ = (np.array(
    [[1, 0, 0, 0, 1, 0], [1, 1, 0, 0, 0, 1], [1, 0, 1, 0, 0, 1],
     [1, 0, 0, 1, 0, 0], [1, 0, 0, 0, 1, 0], [0, 1, 1, 0, 0, 1]],
    dtype=np.int64).T != 0)

_SHIFTS = []
for _d in range(-5, 6):
    _rs = [r for r in range(6) if 0 <= r + _d < 6 and _ADJ[r, r + _d]]
    if _rs:
        _SHIFTS.append((_d, tuple(_rs)))

_NEG = -1e30
_SLOPE = 0.2


def _leaky(v):
    return jnp.where(v >= 0, v, _SLOPE * v)


def _matmul_t(a, w):
    return jax.lax.dot_general(a, w, (((1,), (1,)), ((), ())),
                               preferred_element_type=jnp.float32)


def _channel_onehot(row0, n_rows, period, group):
    rows = row0 + jax.lax.broadcasted_iota(jnp.int32, (n_rows, 1), 0)
    ch = jax.lax.rem(rows, period) // group
    lanes = jax.lax.broadcasted_iota(jnp.int32, (n_rows, 64), 1)
    return (ch == lanes).astype(jnp.float32)


def _pass1_body(nlayer, gb, tv, v_per_g, d_rows,
                xg_ref, xd_ref, xs_ref, wl_ref, bl_ref, wr_ref, br_ref,
                att_ref, b_ref, base_ref, a_ref, stats_ref):
    j = pl.program_id(0)
    xg = xg_ref[...]
    xd = xd_ref[...]
    xs = xs_ref[...]

    nr_b = xg.shape[0]
    nr_a = xd.shape[0]
    rmod = jax.lax.rem(jax.lax.broadcasted_iota(jnp.int32, (nr_a, 1), 0), 6)

    base_acc = jnp.zeros((nr_b, 64), jnp.float32)
    a_acc = jnp.zeros((nr_a, 64), jnp.float32)

    for i in range(nlayer):
        wl = wl_ref[i]
        wr = wr_ref[i]
        bl_i = bl_ref[i:i + 1, :]
        br_i = br_ref[i:i + 1, :]
        att_i = att_ref[i:i + 1, :]
        b_i = b_ref[i:i + 1, :]

        xl = _matmul_t(xg, wl) + bl_i
        s = _matmul_t(xs, wl) + bl_i
        xld = _matmul_t(xd, wl) + bl_i
        xrd = _matmul_t(xd, wr) + br_i

        sl = jnp.sum(_leaky(xld + xrd) * att_i, axis=1, keepdims=True)

        s_roll = []
        e_list = []
        mask_list = []
        for d, rset in _SHIFTS:
            sr = s if d == 0 else jnp.roll(s, -d, axis=0)
            s_roll.append(sr)
            e = jnp.sum(_leaky(sr + xrd) * att_i, axis=1, keepdims=True)
            if len(rset) == 6:
                mask = None
            else:
                mask = functools.reduce(
                    jnp.logical_or, [rmod == r for r in rset])
            e_list.append(e)
            mask_list.append(mask)

        m = sl
        for e, mask in zip(e_list, mask_list):
            em = e if mask is None else jnp.where(mask, e, _NEG)
            m = jnp.maximum(m, em)

        ex_self = jnp.exp(sl - m)
        den = ex_self
        mix = ex_self * xld
        for e, mask, sr in zip(e_list, mask_list, s_roll):
            ex = jnp.exp(e - m)
            if mask is not None:
                ex = jnp.where(mask, ex, 0.0)
            den = den + ex
            mix = mix + ex * sr

        a_acc = a_acc + mix / (den + 1e-16) + b_i
        base_acc = base_acc + xl + b_i

    base_ref[...] = base_acc
    a_ref[...] = a_acc

    row0_b = j * nr_b
    row0_a = j * nr_a
    rows_b = row0_b + jax.lax.broadcasted_iota(jnp.int32, (nr_b, 1), 0)
    tail = (rows_b >= d_rows).astype(jnp.float32)
    rs_b = jnp.sum(base_acc, axis=1, keepdims=True) * tail
    rss_b = jnp.sum(base_acc * base_acc, axis=1, keepdims=True) * tail
    rs_a = jnp.sum(a_acc, axis=1, keepdims=True)
    rss_a = jnp.sum(a_acc * a_acc, axis=1, keepdims=True)

    oh_b = _channel_onehot(row0_b, nr_b, tv, v_per_g)
    oh_a = _channel_onehot(row0_a, nr_a, tv, v_per_g)
    vals_b = jnp.concatenate([rs_b, rss_b], axis=1)
    vals_a = jnp.concatenate([rs_a, rss_a], axis=1)
    contrib = (
        jax.lax.dot_general(oh_b, vals_b, (((0,), (0,)), ((), ())),
                            preferred_element_type=jnp.float32)
        + jax.lax.dot_general(oh_a, vals_a, (((0,), (0,)), ((), ())),
                              preferred_element_type=jnp.float32))

    @pl.when(j == 0)
    def _():
        stats_ref[...] = contrib

    @pl.when(j != 0)
    def _():
        stats_ref[...] = stats_ref[...] + contrib


def _pass2_body(n_total, d_rows, tv, v_per_g,
                base_ref, a_ref, xres_ref, stats_ref, bnw_ref, bnb_ref,
                out_ref):
    j = pl.program_id(0)
    nr = out_ref.shape[0]
    cnt = float(n_total)

    s = stats_ref[:, 0:1]
    ss = stats_ref[:, 1:2]
    mean = s / cnt
    var = ss / cnt - mean * mean
    inv = jax.lax.rsqrt(var + 1e-5)
    scale_c = bnw_ref[...] * inv
    shift_c = bnb_ref[...] - mean * scale_c
    scsh = jnp.concatenate([scale_c, shift_c], axis=1)

    oh = _channel_onehot(j * nr, nr, tv, v_per_g)
    rowsc = jax.lax.dot_general(oh, scsh, (((1,), (0,)), ((), ())),
                                preferred_element_type=jnp.float32)
    scale_r = rowsc[:, 0:1]
    shift_r = rowsc[:, 1:2]

    rows = j * nr + jax.lax.broadcasted_iota(jnp.int32, (nr, 1), 0)
    y = jnp.where(rows < d_rows, a_ref[...], base_ref[...])
    out_ref[...] = jnp.maximum(y * scale_r + shift_r + xres_ref[...], 0.0)


def kernel(x, Wl, bl, Wr, br, att, b, bn_w, bn_b):
    NM, C, T, V = x.shape
    N = NM * T * V
    G = NM * T
    D = 6 * G
    TV = T * V

    resh = jnp.transpose(x, (0, 2, 3, 1)).reshape(N, C)
    srcflat = resh.reshape(G, V, C)[:, :6, :].reshape(6 * G, C)
    xflat = x.reshape(N, C)
    bnw2 = bn_w.reshape(C, 1)
    bnb2 = bn_b.reshape(C, 1)

    GB = 128
    grid1 = G // GB
    nb = V * GB
    na = 6 * GB

    p1 = pl.pallas_call(
        functools.partial(_pass1_body, 3, GB, TV, V, D),
        grid=(grid1,),
        in_specs=[
            pl.BlockSpec((nb, C), lambda j: (j, 0)),
            pl.BlockSpec((na, C), lambda j: (j, 0)),
            pl.BlockSpec((na, C), lambda j: (j, 0)),
            pl.BlockSpec((3, C, C), lambda j: (0, 0, 0)),
            pl.BlockSpec((3, C), lambda j: (0, 0)),
            pl.BlockSpec((3, C, C), lambda j: (0, 0, 0)),
            pl.BlockSpec((3, C), lambda j: (0, 0)),
            pl.BlockSpec((3, C), lambda j: (0, 0)),
            pl.BlockSpec((3, C), lambda j: (0, 0)),
        ],
        out_specs=[
            pl.BlockSpec((nb, C), lambda j: (j, 0)),
            pl.BlockSpec((na, C), lambda j: (j, 0)),
            pl.BlockSpec((C, 2), lambda j: (0, 0)),
        ],
        out_shape=[
            jax.ShapeDtypeStruct((N, C), jnp.float32),
            jax.ShapeDtypeStruct((D, C), jnp.float32),
            jax.ShapeDtypeStruct((C, 2), jnp.float32),
        ],
        compiler_params=pltpu.CompilerParams(
            dimension_semantics=("arbitrary",)),
    )
    base, a_out, stats = p1(resh, resh, srcflat, Wl, bl, Wr, br, att, b)

    R2 = 6144
    grid2 = N // R2
    head_blocks = D // R2

    p2 = pl.pallas_call(
        functools.partial(_pass2_body, N, D, TV, V),
        grid=(grid2,),
        in_specs=[
            pl.BlockSpec((R2, C), lambda j: (j, 0)),
            pl.BlockSpec((R2, C),
                         lambda j: (jnp.minimum(j, head_blocks - 1), 0)),
            pl.BlockSpec((R2, C), lambda j: (j, 0)),
            pl.BlockSpec((C, 2), lambda j: (0, 0)),
            pl.BlockSpec((C, 1), lambda j: (0, 0)),
            pl.BlockSpec((C, 1), lambda j: (0, 0)),
        ],
        out_specs=pl.BlockSpec((R2, C), lambda j: (j, 0)),
        out_shape=jax.ShapeDtypeStruct((N, C), jnp.float32),
        compiler_params=pltpu.CompilerParams(
            dimension_semantics=("arbitrary",)),
    )
    out = p2(base, a_out, xflat, stats, bnw2, bnb2)
    return out.reshape(NM, C, T, V)


# speedup vs baseline: 4.7918x; 1.0938x over previous
import functools

import jax
import jax.numpy as jnp
import numpy as np
from jax.experimental import pallas as pl
from jax.experimental.pallas import tpu as pltpu

_ADJ = (np.array(
    [[1, 0, 0, 0, 1, 0], [1, 1, 0, 0, 0, 1], [1, 0, 1, 0, 0, 1],
     [1, 0, 0, 1, 0, 0], [1, 0, 0, 0, 1, 0], [0, 1, 1, 0, 0, 1]],
    dtype=np.int64).T != 0)

_SHIFTS = []
for _d in range(-5, 6):
    _rs = [r for r in range(6) if 0 <= r + _d < 6 and _ADJ[r, r + _d]]
    if _rs:
        _SHIFTS.append((_d, tuple(_rs)))

_NEG = -1e30
_SLOPE = 0.2


def _leaky(v):
    return jnp.where(v >= 0, v, _SLOPE * v)


def _matmul_t(a, w):
    return jax.lax.dot_general(a, w, (((1,), (1,)), ((), ())),
                               preferred_element_type=jnp.float32)


def _channel_onehot(row0, n_rows, period, group):
    rows = row0 + jax.lax.broadcasted_iota(jnp.int32, (n_rows, 1), 0)
    ch = jax.lax.rem(rows, period) // group
    lanes = jax.lax.broadcasted_iota(jnp.int32, (n_rows, 64), 1)
    return (ch == lanes).astype(jnp.float32)


def _pass0_body(nm_per_step, tv, v_per_g, xn_ref, resh_ref, src_ref):
    for k in range(nm_per_step):
        tr = jnp.transpose(xn_ref[k])
        resh_ref[k * tv:(k + 1) * tv, :] = tr
        for c in range(6):
            sc = resh_ref[pl.Slice(k * tv + c, tv // v_per_g, v_per_g), :]
            src_ref[pl.Slice(k * (tv // v_per_g) * 6 + c,
                             tv // v_per_g, 6), :] = sc


def _pass1_body(nlayer, gb, tv, v_per_g, d_rows,
                xg_ref, xd_ref, xs_ref, wl_ref, bl_ref, wr_ref, br_ref,
                att_ref, b_ref, base_ref, a_ref, stats_ref):
    j = pl.program_id(0)
    xg = xg_ref[...]
    xd = xd_ref[...]
    xs = xs_ref[...]

    nr_b = xg.shape[0]
    nr_a = xd.shape[0]
    rmod = jax.lax.rem(jax.lax.broadcasted_iota(jnp.int32, (nr_a, 1), 0), 6)

    base_acc = jnp.zeros((nr_b, 64), jnp.float32)
    a_acc = jnp.zeros((nr_a, 64), jnp.float32)

    for i in range(nlayer):
        wl = wl_ref[i]
        wr = wr_ref[i]
        bl_i = bl_ref[i:i + 1, :]
        br_i = br_ref[i:i + 1, :]
        att_i = att_ref[i:i + 1, :]
        b_i = b_ref[i:i + 1, :]

        xl = _matmul_t(xg, wl) + bl_i
        s = _matmul_t(xs, wl) + bl_i
        xld = _matmul_t(xd, wl) + bl_i
        xrd = _matmul_t(xd, wr) + br_i

        sl = jnp.sum(_leaky(xld + xrd) * att_i, axis=1, keepdims=True)

        s_roll = []
        e_list = []
        mask_list = []
        for d, rset in _SHIFTS:
            sr = s if d == 0 else jnp.roll(s, -d, axis=0)
            s_roll.append(sr)
            e = jnp.sum(_leaky(sr + xrd) * att_i, axis=1, keepdims=True)
            if len(rset) == 6:
                mask = None
            else:
                mask = functools.reduce(
                    jnp.logical_or, [rmod == r for r in rset])
            e_list.append(e)
            mask_list.append(mask)

        m = sl
        for e, mask in zip(e_list, mask_list):
            em = e if mask is None else jnp.where(mask, e, _NEG)
            m = jnp.maximum(m, em)

        ex_self = jnp.exp(sl - m)
        den = ex_self
        mix = ex_self * xld
        for e, mask, sr in zip(e_list, mask_list, s_roll):
            ex = jnp.exp(e - m)
            if mask is not None:
                ex = jnp.where(mask, ex, 0.0)
            den = den + ex
            mix = mix + ex * sr

        a_acc = a_acc + mix / (den + 1e-16) + b_i
        base_acc = base_acc + xl + b_i

    base_ref[...] = base_acc
    a_ref[...] = a_acc

    row0_b = j * nr_b
    row0_a = j * nr_a
    rows_b = row0_b + jax.lax.broadcasted_iota(jnp.int32, (nr_b, 1), 0)
    tail = (rows_b >= d_rows).astype(jnp.float32)
    rs_b = jnp.sum(base_acc, axis=1, keepdims=True) * tail
    rss_b = jnp.sum(base_acc * base_acc, axis=1, keepdims=True) * tail
    rs_a = jnp.sum(a_acc, axis=1, keepdims=True)
    rss_a = jnp.sum(a_acc * a_acc, axis=1, keepdims=True)

    oh_b = _channel_onehot(row0_b, nr_b, tv, v_per_g)
    oh_a = _channel_onehot(row0_a, nr_a, tv, v_per_g)
    vals_b = jnp.concatenate([rs_b, rss_b], axis=1)
    vals_a = jnp.concatenate([rs_a, rss_a], axis=1)
    contrib = (
        jax.lax.dot_general(oh_b, vals_b, (((0,), (0,)), ((), ())),
                            preferred_element_type=jnp.float32)
        + jax.lax.dot_general(oh_a, vals_a, (((0,), (0,)), ((), ())),
                              preferred_element_type=jnp.float32))

    @pl.when(j == 0)
    def _():
        stats_ref[...] = contrib

    @pl.when(j != 0)
    def _():
        stats_ref[...] = stats_ref[...] + contrib


def _pass2_body(n_total, d_rows, tv, v_per_g,
                base_ref, a_ref, xres_ref, stats_ref, bnw_ref, bnb_ref,
                out_ref):
    j = pl.program_id(0)
    nr = out_ref.shape[0]
    cnt = float(n_total)

    s = stats_ref[:, 0:1]
    ss = stats_ref[:, 1:2]
    mean = s / cnt
    var = ss / cnt - mean * mean
    inv = jax.lax.rsqrt(var + 1e-5)
    scale_c = bnw_ref[...] * inv
    shift_c = bnb_ref[...] - mean * scale_c
    scsh = jnp.concatenate([scale_c, shift_c], axis=1)

    oh = _channel_onehot(j * nr, nr, tv, v_per_g)
    rowsc = jax.lax.dot_general(oh, scsh, (((1,), (0,)), ((), ())),
                                preferred_element_type=jnp.float32)
    scale_r = rowsc[:, 0:1]
    shift_r = rowsc[:, 1:2]

    rows = j * nr + jax.lax.broadcasted_iota(jnp.int32, (nr, 1), 0)
    y = jnp.where(rows < d_rows, a_ref[...], base_ref[...])
    out_ref[...] = jnp.maximum(y * scale_r + shift_r + xres_ref[...], 0.0)


def kernel(x, Wl, bl, Wr, br, att, b, bn_w, bn_b):
    NM, C, T, V = x.shape
    N = NM * T * V
    G = NM * T
    D = 6 * G
    TV = T * V

    xnat = x.reshape(NM, C, TV)
    xflat = x.reshape(N, C)
    bnw2 = bn_w.reshape(C, 1)
    bnb2 = bn_b.reshape(C, 1)

    NMS = 8
    p0 = pl.pallas_call(
        functools.partial(_pass0_body, NMS, TV, V),
        grid=(NM // NMS,),
        in_specs=[pl.BlockSpec((NMS, C, TV), lambda j: (j, 0, 0))],
        out_specs=[
            pl.BlockSpec((NMS * TV, C), lambda j: (j, 0)),
            pl.BlockSpec((NMS * 6 * T, C), lambda j: (j, 0)),
        ],
        out_shape=[
            jax.ShapeDtypeStruct((N, C), jnp.float32),
            jax.ShapeDtypeStruct((D, C), jnp.float32),
        ],
        compiler_params=pltpu.CompilerParams(
            dimension_semantics=("arbitrary",)),
    )
    resh, srcflat = p0(xnat)

    GB = 128
    grid1 = G // GB
    nb = V * GB
    na = 6 * GB

    p1 = pl.pallas_call(
        functools.partial(_pass1_body, 3, GB, TV, V, D),
        grid=(grid1,),
        in_specs=[
            pl.BlockSpec((nb, C), lambda j: (j, 0)),
            pl.BlockSpec((na, C), lambda j: (j, 0)),
            pl.BlockSpec((na, C), lambda j: (j, 0)),
            pl.BlockSpec((3, C, C), lambda j: (0, 0, 0)),
            pl.BlockSpec((3, C), lambda j: (0, 0)),
            pl.BlockSpec((3, C, C), lambda j: (0, 0, 0)),
            pl.BlockSpec((3, C), lambda j: (0, 0)),
            pl.BlockSpec((3, C), lambda j: (0, 0)),
            pl.BlockSpec((3, C), lambda j: (0, 0)),
        ],
        out_specs=[
            pl.BlockSpec((nb, C), lambda j: (j, 0)),
            pl.BlockSpec((na, C), lambda j: (j, 0)),
            pl.BlockSpec((C, 2), lambda j: (0, 0)),
        ],
        out_shape=[
            jax.ShapeDtypeStruct((N, C), jnp.float32),
            jax.ShapeDtypeStruct((D, C), jnp.float32),
            jax.ShapeDtypeStruct((C, 2), jnp.float32),
        ],
        compiler_params=pltpu.CompilerParams(
            dimension_semantics=("arbitrary",)),
    )
    base, a_out, stats = p1(resh, resh, srcflat, Wl, bl, Wr, br, att, b)

    R2 = 6144
    grid2 = N // R2
    head_blocks = D // R2

    p2 = pl.pallas_call(
        functools.partial(_pass2_body, N, D, TV, V),
        grid=(grid2,),
        in_specs=[
            pl.BlockSpec((R2, C), lambda j: (j, 0)),
            pl.BlockSpec((R2, C),
                         lambda j: (jnp.minimum(j, head_blocks - 1), 0)),
            pl.BlockSpec((R2, C), lambda j: (j, 0)),
            pl.BlockSpec((C, 2), lambda j: (0, 0)),
            pl.BlockSpec((C, 1), lambda j: (0, 0)),
            pl.BlockSpec((C, 1), lambda j: (0, 0)),
        ],
        out_specs=pl.BlockSpec((R2, C), lambda j: (j, 0)),
        out_shape=jax.ShapeDtypeStruct((N, C), jnp.float32),
        compiler_params=pltpu.CompilerParams(
            dimension_semantics=("arbitrary",)),
    )
    out = p2(base, a_out, xflat, stats, bnw2, bnb2)
    return out.reshape(NM, C, T, V)


# speedup vs baseline: 6.4644x; 1.3490x over previous
import functools

import jax
import jax.numpy as jnp
import numpy as np
from jax.experimental import pallas as pl
from jax.experimental.pallas import tpu as pltpu

_ADJ = (np.array(
    [[1, 0, 0, 0, 1, 0], [1, 1, 0, 0, 0, 1], [1, 0, 1, 0, 0, 1],
     [1, 0, 0, 1, 0, 0], [1, 0, 0, 0, 1, 0], [0, 1, 1, 0, 0, 1]],
    dtype=np.int64).T != 0)

_NBRS = [tuple(c for c in range(6) if _ADJ[r, c]) for r in range(6)]

_SLOPE = 0.2


def _leaky(v):
    return jnp.where(v >= 0, v, _SLOPE * v)


def _matmul_t(a, w):
    return jax.lax.dot_general(a, w, (((1,), (1,)), ((), ())),
                               preferred_element_type=jnp.float32)


def _channel_onehot(row0, n_rows, period, group):
    rows = row0 + jax.lax.broadcasted_iota(jnp.int32, (n_rows, 1), 0)
    ch = jax.lax.rem(rows, period) // group
    lanes = jax.lax.broadcasted_iota(jnp.int32, (n_rows, 64), 1)
    return (ch == lanes).astype(jnp.float32)


def _pass0_body(nm_per_step, tv, v_per_g, c_dim, xn_ref, resh_ref, src_ref,
                xflat_ref):
    for k in range(nm_per_step):
        xn = xn_ref[k]
        tr = jnp.transpose(xn)
        resh_ref[k * tv:(k + 1) * tv, :] = tr
        for c in range(6):
            sc = resh_ref[pl.Slice(k * tv + c, tv // v_per_g, v_per_g), :]
            src_ref[pl.Slice(k * (tv // v_per_g) * 6 + c,
                             tv // v_per_g, 6), :] = sc
        for r in range(v_per_g):
            xflat_ref[pl.Slice(k * tv + r, tv // v_per_g, v_per_g), :] = (
                xn[:, r * c_dim:(r + 1) * c_dim])


def _pass1_body(nlayer, gb, tv, v_per_g, d_rows,
                xg_ref, xd_ref, xs_ref, wl_ref, bl_ref, wr_ref, br_ref,
                att_ref, b_ref, base_ref, a_ref, stats_ref,
                s_scr, xld_scr, xrd_scr, a_scr):
    j = pl.program_id(0)
    xg = xg_ref[...]
    xd = xd_ref[...]
    xs = xs_ref[...]

    nr_b = xg.shape[0]
    nr_a = xd.shape[0]
    ng = nr_a // 6

    base_acc = jnp.zeros((nr_b, 64), jnp.float32)
    a_scr[...] = jnp.zeros((nr_a, 64), jnp.float32)

    for i in range(nlayer):
        wl = wl_ref[i]
        wr = wr_ref[i]
        bl_i = bl_ref[i:i + 1, :]
        br_i = br_ref[i:i + 1, :]
        att_i = att_ref[i:i + 1, :]
        b_i = b_ref[i:i + 1, :]

        xl = _matmul_t(xg, wl) + bl_i
        s_scr[...] = _matmul_t(xs, wl) + bl_i
        xld_scr[...] = _matmul_t(xd, wl) + bl_i
        xrd_scr[...] = _matmul_t(xd, wr) + br_i

        for r in range(6):
            dl_r = xld_scr[pl.Slice(r, ng, 6), :]
            dr_r = xrd_scr[pl.Slice(r, ng, 6), :]
            m = jnp.sum(_leaky(dl_r + dr_r) * att_i, axis=1, keepdims=True)
            sl_r = m
            s_list = []
            e_list = []
            for c in _NBRS[r]:
                sc = s_scr[pl.Slice(c, ng, 6), :]
                e = jnp.sum(_leaky(sc + dr_r) * att_i, axis=1, keepdims=True)
                s_list.append(sc)
                e_list.append(e)
                m = jnp.maximum(m, e)
            ex_self = jnp.exp(sl_r - m)
            den = ex_self
            mix = ex_self * dl_r
            for sc, e in zip(s_list, e_list):
                ex = jnp.exp(e - m)
                den = den + ex
                mix = mix + ex * sc
            sl_idx = pl.Slice(r, ng, 6)
            a_scr[sl_idx, :] = (a_scr[sl_idx, :]
                                + mix / (den + 1e-16) + b_i)

        base_acc = base_acc + xl + b_i

    a_acc = a_scr[...]
    base_ref[...] = base_acc
    a_ref[...] = a_acc

    row0_b = j * nr_b
    row0_a = j * nr_a
    rows_b = row0_b + jax.lax.broadcasted_iota(jnp.int32, (nr_b, 1), 0)
    tail = (rows_b >= d_rows).astype(jnp.float32)
    rs_b = jnp.sum(base_acc, axis=1, keepdims=True) * tail
    rss_b = jnp.sum(base_acc * base_acc, axis=1, keepdims=True) * tail
    rs_a = jnp.sum(a_acc, axis=1, keepdims=True)
    rss_a = jnp.sum(a_acc * a_acc, axis=1, keepdims=True)

    oh_b = _channel_onehot(row0_b, nr_b, tv, v_per_g)
    oh_a = _channel_onehot(row0_a, nr_a, tv, v_per_g)
    vals_b = jnp.concatenate([rs_b, rss_b], axis=1)
    vals_a = jnp.concatenate([rs_a, rss_a], axis=1)
    contrib = (
        jax.lax.dot_general(oh_b, vals_b, (((0,), (0,)), ((), ())),
                            preferred_element_type=jnp.float32)
        + jax.lax.dot_general(oh_a, vals_a, (((0,), (0,)), ((), ())),
                              preferred_element_type=jnp.float32))

    @pl.when(j == 0)
    def _():
        stats_ref[...] = contrib

    @pl.when(j != 0)
    def _():
        stats_ref[...] = stats_ref[...] + contrib


def _pass2_body(n_total, d_rows, tv, v_per_g,
                base_ref, a_ref, xres_ref, stats_ref, bnw_ref, bnb_ref,
                out_ref):
    j = pl.program_id(0)
    nr = out_ref.shape[0]
    cnt = float(n_total)

    s = stats_ref[:, 0:1]
    ss = stats_ref[:, 1:2]
    mean = s / cnt
    var = ss / cnt - mean * mean
    inv = jax.lax.rsqrt(var + 1e-5)
    scale_c = bnw_ref[...] * inv
    shift_c = bnb_ref[...] - mean * scale_c
    scsh = jnp.concatenate([scale_c, shift_c], axis=1)

    oh = _channel_onehot(j * nr, nr, tv, v_per_g)
    rowsc = jax.lax.dot_general(oh, scsh, (((1,), (0,)), ((), ())),
                                preferred_element_type=jnp.float32)
    scale_r = rowsc[:, 0:1]
    shift_r = rowsc[:, 1:2]

    rows = j * nr + jax.lax.broadcasted_iota(jnp.int32, (nr, 1), 0)
    y = jnp.where(rows < d_rows, a_ref[...], base_ref[...])
    out_ref[...] = jnp.maximum(y * scale_r + shift_r + xres_ref[...], 0.0)


def kernel(x, Wl, bl, Wr, br, att, b, bn_w, bn_b):
    NM, C, T, V = x.shape
    N = NM * T * V
    G = NM * T
    D = 6 * G
    TV = T * V

    xnat = x.reshape(NM, C, TV)
    bnw2 = bn_w.reshape(C, 1)
    bnb2 = bn_b.reshape(C, 1)

    NMS = 8
    p0 = pl.pallas_call(
        functools.partial(_pass0_body, NMS, TV, V, C),
        grid=(NM // NMS,),
        in_specs=[pl.BlockSpec((NMS, C, TV), lambda j: (j, 0, 0))],
        out_specs=[
            pl.BlockSpec((NMS * TV, C), lambda j: (j, 0)),
            pl.BlockSpec((NMS * 6 * T, C), lambda j: (j, 0)),
            pl.BlockSpec((NMS * TV, C), lambda j: (j, 0)),
        ],
        out_shape=[
            jax.ShapeDtypeStruct((N, C), jnp.float32),
            jax.ShapeDtypeStruct((D, C), jnp.float32),
            jax.ShapeDtypeStruct((N, C), jnp.float32),
        ],
        compiler_params=pltpu.CompilerParams(
            dimension_semantics=("arbitrary",)),
    )
    resh, srcflat, xflat = p0(xnat)

    GB = 256
    grid1 = G // GB
    nb = V * GB
    na = 6 * GB

    p1 = pl.pallas_call(
        functools.partial(_pass1_body, 3, GB, TV, V, D),
        grid=(grid1,),
        in_specs=[
            pl.BlockSpec((nb, C), lambda j: (j, 0)),
            pl.BlockSpec((na, C), lambda j: (j, 0)),
            pl.BlockSpec((na, C), lambda j: (j, 0)),
            pl.BlockSpec((3, C, C), lambda j: (0, 0, 0)),
            pl.BlockSpec((3, C), lambda j: (0, 0)),
            pl.BlockSpec((3, C, C), lambda j: (0, 0, 0)),
            pl.BlockSpec((3, C), lambda j: (0, 0)),
            pl.BlockSpec((3, C), lambda j: (0, 0)),
            pl.BlockSpec((3, C), lambda j: (0, 0)),
        ],
        out_specs=[
            pl.BlockSpec((nb, C), lambda j: (j, 0)),
            pl.BlockSpec((na, C), lambda j: (j, 0)),
            pl.BlockSpec((C, 2), lambda j: (0, 0)),
        ],
        out_shape=[
            jax.ShapeDtypeStruct((N, C), jnp.float32),
            jax.ShapeDtypeStruct((D, C), jnp.float32),
            jax.ShapeDtypeStruct((C, 2), jnp.float32),
        ],
        scratch_shapes=[
            pltpu.VMEM((na, C), jnp.float32),
            pltpu.VMEM((na, C), jnp.float32),
            pltpu.VMEM((na, C), jnp.float32),
            pltpu.VMEM((na, C), jnp.float32),
        ],
        compiler_params=pltpu.CompilerParams(
            dimension_semantics=("arbitrary",)),
    )
    base, a_out, stats = p1(resh, resh, srcflat, Wl, bl, Wr, br, att, b)

    R2 = 6144
    grid2 = N // R2
    head_blocks = D // R2

    p2 = pl.pallas_call(
        functools.partial(_pass2_body, N, D, TV, V),
        grid=(grid2,),
        in_specs=[
            pl.BlockSpec((R2, C), lambda j: (j, 0)),
            pl.BlockSpec((R2, C),
                         lambda j: (jnp.minimum(j, head_blocks - 1), 0)),
            pl.BlockSpec((R2, C), lambda j: (j, 0)),
            pl.BlockSpec((C, 2), lambda j: (0, 0)),
            pl.BlockSpec((C, 1), lambda j: (0, 0)),
            pl.BlockSpec((C, 1), lambda j: (0, 0)),
        ],
        out_specs=pl.BlockSpec((R2, C), lambda j: (j, 0)),
        out_shape=jax.ShapeDtypeStruct((N, C), jnp.float32),
        compiler_params=pltpu.CompilerParams(
            dimension_semantics=("arbitrary",)),
    )
    out = p2(base, a_out, xflat, stats, bnw2, bnb2)
    return out.reshape(NM, C, T, V)


# speedup vs baseline: 7.3765x; 1.1411x over previous
import functools

import jax
import jax.numpy as jnp
import numpy as np
from jax.experimental import pallas as pl
from jax.experimental.pallas import tpu as pltpu

_ADJ = (np.array(
    [[1, 0, 0, 0, 1, 0], [1, 1, 0, 0, 0, 1], [1, 0, 1, 0, 0, 1],
     [1, 0, 0, 1, 0, 0], [1, 0, 0, 0, 1, 0], [0, 1, 1, 0, 0, 1]],
    dtype=np.int64).T != 0)

_NBRS = [tuple(c for c in range(6) if _ADJ[r, c]) for r in range(6)]

_SLOPE = 0.2


def _leaky(v):
    return jnp.where(v >= 0, v, _SLOPE * v)


def _matmul_t(a, w):
    return jax.lax.dot_general(a, w, (((1,), (1,)), ((), ())),
                               preferred_element_type=jnp.float32)


def _onehot_np(row0, n_rows, period, group):
    ch = ((row0 + np.arange(n_rows)) % period) // group
    return (ch[:, None] == np.arange(64)[None, :]).astype(np.float32)


def _pass0_body(nm_per_step, tv, v_per_g, c_dim, xn_ref, resh_ref, src_ref,
                xflat_ref):
    for k in range(nm_per_step):
        xn = xn_ref[k]
        tr = jnp.transpose(xn)
        resh_ref[k * tv:(k + 1) * tv, :] = tr
        for c in range(6):
            sc = resh_ref[pl.Slice(k * tv + c, tv // v_per_g, v_per_g), :]
            src_ref[pl.Slice(k * (tv // v_per_g) * 6 + c,
                             tv // v_per_g, 6), :] = sc
        for r in range(v_per_g):
            xflat_ref[pl.Slice(k * tv + r, tv // v_per_g, v_per_g), :] = (
                xn[:, r * c_dim:(r + 1) * c_dim])


def _pass1_body(nlayer, gb, tv, v_per_g, d_rows,
                xg_ref, xd_ref, xs_ref, wl_ref, bl_ref, wr_ref, br_ref,
                att_ref, b_ref, ohb_ref, oha_ref,
                base_ref, a_ref, stats_ref,
                s_scr, xld_scr, xrd_scr, a_scr):
    j = pl.program_id(0)
    xg = xg_ref[...]
    xd = xd_ref[...]
    xs = xs_ref[...]

    nr_b = xg.shape[0]
    nr_a = xd.shape[0]
    ng = nr_a // 6

    base_acc = jnp.zeros((nr_b, 64), jnp.float32)
    a_scr[...] = jnp.zeros((nr_a, 64), jnp.float32)

    for i in range(nlayer):
        wl = wl_ref[i]
        wr = wr_ref[i]
        bl_i = bl_ref[i:i + 1, :]
        br_i = br_ref[i:i + 1, :]
        att_i = att_ref[i:i + 1, :]
        b_i = b_ref[i:i + 1, :]

        xl = _matmul_t(xg, wl) + bl_i
        s_scr[...] = _matmul_t(xs, wl) + bl_i
        xld_scr[...] = _matmul_t(xd, wl) + bl_i
        xrd_scr[...] = _matmul_t(xd, wr) + br_i

        for r in range(6):
            dl_r = xld_scr[pl.Slice(r, ng, 6), :]
            dr_r = xrd_scr[pl.Slice(r, ng, 6), :]
            m = jnp.sum(_leaky(dl_r + dr_r) * att_i, axis=1, keepdims=True)
            sl_r = m
            s_list = []
            e_list = []
            for c in _NBRS[r]:
                sc = s_scr[pl.Slice(c, ng, 6), :]
                e = jnp.sum(_leaky(sc + dr_r) * att_i, axis=1, keepdims=True)
                s_list.append(sc)
                e_list.append(e)
                m = jnp.maximum(m, e)
            ex_self = jnp.exp(sl_r - m)
            den = ex_self
            mix = ex_self * dl_r
            for sc, e in zip(s_list, e_list):
                ex = jnp.exp(e - m)
                den = den + ex
                mix = mix + ex * sc
            sl_idx = pl.Slice(r, ng, 6)
            a_scr[sl_idx, :] = (a_scr[sl_idx, :]
                                + mix / (den + 1e-16) + b_i)

        base_acc = base_acc + xl + b_i

    a_acc = a_scr[...]
    base_ref[...] = base_acc
    a_ref[...] = a_acc

    rows_b = (j * nr_b
              + jax.lax.broadcasted_iota(jnp.int32, (nr_b, 1), 0))
    tail = (rows_b >= d_rows).astype(jnp.float32)
    rs_b = jnp.sum(base_acc, axis=1, keepdims=True) * tail
    rss_b = jnp.sum(base_acc * base_acc, axis=1, keepdims=True) * tail
    rs_a = jnp.sum(a_acc, axis=1, keepdims=True)
    rss_a = jnp.sum(a_acc * a_acc, axis=1, keepdims=True)

    oh_b = ohb_ref[...]
    oh_a = oha_ref[...]
    vals_b = jnp.concatenate([rs_b, rss_b], axis=1)
    vals_a = jnp.concatenate([rs_a, rss_a], axis=1)
    contrib = (
        jax.lax.dot_general(oh_b, vals_b, (((0,), (0,)), ((), ())),
                            preferred_element_type=jnp.float32)
        + jax.lax.dot_general(oh_a, vals_a, (((0,), (0,)), ((), ())),
                              preferred_element_type=jnp.float32))

    @pl.when(j == 0)
    def _():
        stats_ref[...] = contrib

    @pl.when(j != 0)
    def _():
        stats_ref[...] = stats_ref[...] + contrib


def _pass2_body(n_total, d_rows, tv, v_per_g,
                base_ref, a_ref, xres_ref, stats_ref, bnw_ref, bnb_ref,
                oh_ref, out_ref):
    j = pl.program_id(0)
    nr = out_ref.shape[0]
    cnt = float(n_total)

    s = stats_ref[:, 0:1]
    ss = stats_ref[:, 1:2]
    mean = s / cnt
    var = ss / cnt - mean * mean
    inv = jax.lax.rsqrt(var + 1e-5)
    scale_c = bnw_ref[...] * inv
    shift_c = bnb_ref[...] - mean * scale_c
    scsh = jnp.concatenate([scale_c, shift_c], axis=1)

    oh = oh_ref[...]
    rowsc = jax.lax.dot_general(oh, scsh, (((1,), (0,)), ((), ())),
                                preferred_element_type=jnp.float32)
    scale_r = rowsc[:, 0:1]
    shift_r = rowsc[:, 1:2]

    rows = j * nr + jax.lax.broadcasted_iota(jnp.int32, (nr, 1), 0)
    y = jnp.where(rows < d_rows, a_ref[...], base_ref[...])
    out_ref[...] = jnp.maximum(y * scale_r + shift_r + xres_ref[...], 0.0)


def kernel(x, Wl, bl, Wr, br, att, b, bn_w, bn_b):
    NM, C, T, V = x.shape
    N = NM * T * V
    G = NM * T
    D = 6 * G
    TV = T * V

    xnat = x.reshape(NM, C, TV)
    bnw2 = bn_w.reshape(C, 1)
    bnb2 = bn_b.reshape(C, 1)

    NMS = 8
    p0 = pl.pallas_call(
        functools.partial(_pass0_body, NMS, TV, V, C),
        grid=(NM // NMS,),
        in_specs=[pl.BlockSpec((NMS, C, TV), lambda j: (j, 0, 0))],
        out_specs=[
            pl.BlockSpec((NMS * TV, C), lambda j: (j, 0)),
            pl.BlockSpec((NMS * 6 * T, C), lambda j: (j, 0)),
            pl.BlockSpec((NMS * TV, C), lambda j: (j, 0)),
        ],
        out_shape=[
            jax.ShapeDtypeStruct((N, C), jnp.float32),
            jax.ShapeDtypeStruct((D, C), jnp.float32),
            jax.ShapeDtypeStruct((N, C), jnp.float32),
        ],
        compiler_params=pltpu.CompilerParams(
            dimension_semantics=("arbitrary",)),
    )
    resh, srcflat, xflat = p0(xnat)

    GB = 256
    grid1 = G // GB
    nb = V * GB
    na = 6 * GB

    ohb = jnp.asarray(_onehot_np(0, nb, TV, V))
    oha = jnp.asarray(np.concatenate(
        [_onehot_np(jj * na, na, TV, V) for jj in range(7)], axis=0))

    p1 = pl.pallas_call(
        functools.partial(_pass1_body, 3, GB, TV, V, D),
        grid=(grid1,),
        in_specs=[
            pl.BlockSpec((nb, C), lambda j: (j, 0)),
            pl.BlockSpec((na, C), lambda j: (j, 0)),
            pl.BlockSpec((na, C), lambda j: (j, 0)),
            pl.BlockSpec((3, C, C), lambda j: (0, 0, 0)),
            pl.BlockSpec((3, C), lambda j: (0, 0)),
            pl.BlockSpec((3, C, C), lambda j: (0, 0, 0)),
            pl.BlockSpec((3, C), lambda j: (0, 0)),
            pl.BlockSpec((3, C), lambda j: (0, 0)),
            pl.BlockSpec((3, C), lambda j: (0, 0)),
            pl.BlockSpec((nb, C), lambda j: (0, 0)),
            pl.BlockSpec((na, C), lambda j: (jax.lax.rem(j, 7), 0)),
        ],
        out_specs=[
            pl.BlockSpec((nb, C), lambda j: (j, 0)),
            pl.BlockSpec((na, C), lambda j: (j, 0)),
            pl.BlockSpec((C, 2), lambda j: (0, 0)),
        ],
        out_shape=[
            jax.ShapeDtypeStruct((N, C), jnp.float32),
            jax.ShapeDtypeStruct((35 * na, C), jnp.float32),
            jax.ShapeDtypeStruct((C, 2), jnp.float32),
        ],
        scratch_shapes=[
            pltpu.VMEM((na, C), jnp.float32),
            pltpu.VMEM((na, C), jnp.float32),
            pltpu.VMEM((na, C), jnp.float32),
            pltpu.VMEM((na, C), jnp.float32),
        ],
        compiler_params=pltpu.CompilerParams(
            dimension_semantics=("arbitrary",)),
    )
    base, a_out, stats = p1(resh, resh, srcflat, Wl, bl, Wr, br, att, b,
                            ohb, oha)

    R2 = 4 * TV
    grid2 = N // R2
    a_blocks = a_out.shape[0] // R2
    oh2 = jnp.asarray(_onehot_np(0, R2, TV, V))

    p2 = pl.pallas_call(
        functools.partial(_pass2_body, N, D, TV, V),
        grid=(grid2,),
        in_specs=[
            pl.BlockSpec((R2, C), lambda j: (j, 0)),
            pl.BlockSpec((R2, C),
                         lambda j: (jnp.minimum(j, a_blocks - 1), 0)),
            pl.BlockSpec((R2, C), lambda j: (j, 0)),
            pl.BlockSpec((C, 2), lambda j: (0, 0)),
            pl.BlockSpec((C, 1), lambda j: (0, 0)),
            pl.BlockSpec((C, 1), lambda j: (0, 0)),
            pl.BlockSpec((R2, C), lambda j: (0, 0)),
        ],
        out_specs=pl.BlockSpec((R2, C), lambda j: (j, 0)),
        out_shape=jax.ShapeDtypeStruct((N, C), jnp.float32),
        compiler_params=pltpu.CompilerParams(
            dimension_semantics=("arbitrary",)),
    )
    out = p2(base, a_out, xflat, stats, bnw2, bnb2, oh2)
    return out.reshape(NM, C, T, V)


# speedup vs baseline: 7.5875x; 1.0286x over previous
import functools

import jax
import jax.numpy as jnp
import numpy as np
from jax.experimental import pallas as pl
from jax.experimental.pallas import tpu as pltpu

_ADJ = (np.array(
    [[1, 0, 0, 0, 1, 0], [1, 1, 0, 0, 0, 1], [1, 0, 1, 0, 0, 1],
     [1, 0, 0, 1, 0, 0], [1, 0, 0, 0, 1, 0], [0, 1, 1, 0, 0, 1]],
    dtype=np.int64).T != 0)

_NBRS = [tuple(c for c in range(6) if _ADJ[r, c]) for r in range(6)]

_SLOPE = 0.2


def _leaky(v):
    return jnp.where(v >= 0, v, _SLOPE * v)


def _matmul_t(a, w):
    return jax.lax.dot_general(a, w, (((1,), (1,)), ((), ())),
                               preferred_element_type=jnp.float32)


def _onehot_np(row0, n_rows, period, group):
    ch = ((row0 + np.arange(n_rows)) % period) // group
    return (ch[:, None] == np.arange(64)[None, :]).astype(np.float32)


def _pass0_body(v_per_g, t_per_step, xt_ref, resh_ref, src_ref):
    for tt in range(t_per_step):
        for v in range(v_per_g):
            tv = jnp.transpose(xt_ref[:, v, tt, :])
            resh_ref[:, tt * v_per_g + v, :] = tv
            if v < 6:
                src_ref[:, tt, v, :] = tv


def _pass1_body(nlayer, gb, tv, v_per_g, d_rows,
                xg_ref, xd_ref, xs_ref, wl_ref, bl_ref, wr_ref, br_ref,
                att_ref, b_ref, ohb_ref, oha_ref,
                base_ref, a_ref, stats_ref,
                s_scr, xld_scr, xrd_scr, a_scr):
    j = pl.program_id(0)
    xg = xg_ref[...]
    xd = xd_ref[...]
    xs = xs_ref[...]

    nr_b = xg.shape[0]
    nr_a = xd.shape[0]
    ng = nr_a // 6

    base_acc = jnp.zeros((nr_b, 64), jnp.float32)
    a_scr[...] = jnp.zeros((nr_a, 64), jnp.float32)

    for i in range(nlayer):
        wl = wl_ref[i]
        wr = wr_ref[i]
        bl_i = bl_ref[i:i + 1, :]
        br_i = br_ref[i:i + 1, :]
        att_i = att_ref[i:i + 1, :]
        b_i = b_ref[i:i + 1, :]

        xl = _matmul_t(xg, wl) + bl_i
        s_scr[...] = _matmul_t(xs, wl) + bl_i
        xld_scr[...] = _matmul_t(xd, wl) + bl_i
        xrd_scr[...] = _matmul_t(xd, wr) + br_i

        for r in range(6):
            dl_r = xld_scr[pl.Slice(r, ng, 6), :]
            dr_r = xrd_scr[pl.Slice(r, ng, 6), :]
            m = jnp.sum(_leaky(dl_r + dr_r) * att_i, axis=1, keepdims=True)
            sl_r = m
            s_list = []
            e_list = []
            for c in _NBRS[r]:
                sc = s_scr[pl.Slice(c, ng, 6), :]
                e = jnp.sum(_leaky(sc + dr_r) * att_i, axis=1, keepdims=True)
                s_list.append(sc)
                e_list.append(e)
                m = jnp.maximum(m, e)
            ex_self = jnp.exp(sl_r - m)
            den = ex_self
            mix = ex_self * dl_r
            for sc, e in zip(s_list, e_list):
                ex = jnp.exp(e - m)
                den = den + ex
                mix = mix + ex * sc
            sl_idx = pl.Slice(r, ng, 6)
            a_scr[sl_idx, :] = (a_scr[sl_idx, :]
                                + mix / (den + 1e-16) + b_i)

        base_acc = base_acc + xl + b_i

    a_acc = a_scr[...]
    base_ref[...] = base_acc
    a_ref[...] = a_acc

    rows_b = (j * nr_b
              + jax.lax.broadcasted_iota(jnp.int32, (nr_b, 1), 0))
    tail = (rows_b >= d_rows).astype(jnp.float32)
    rs_b = jnp.sum(base_acc, axis=1, keepdims=True) * tail
    rss_b = jnp.sum(base_acc * base_acc, axis=1, keepdims=True) * tail
    rs_a = jnp.sum(a_acc, axis=1, keepdims=True)
    rss_a = jnp.sum(a_acc * a_acc, axis=1, keepdims=True)

    oh_b = ohb_ref[...]
    oh_a = oha_ref[...]
    vals_b = jnp.concatenate([rs_b, rss_b], axis=1)
    vals_a = jnp.concatenate([rs_a, rss_a], axis=1)
    contrib = (
        jax.lax.dot_general(oh_b, vals_b, (((0,), (0,)), ((), ())),
                            preferred_element_type=jnp.float32)
        + jax.lax.dot_general(oh_a, vals_a, (((0,), (0,)), ((), ())),
                              preferred_element_type=jnp.float32))

    @pl.when(j == 0)
    def _():
        stats_ref[...] = contrib

    @pl.when(j != 0)
    def _():
        stats_ref[...] = stats_ref[...] + contrib


def _pass2_body(n_total, d_rows, tv, v_per_g, c_dim,
                base_ref, a_ref, resh_ref, stats_ref, bnw_ref, bnb_ref,
                oh_ref, out_ref, xres_scr):
    j = pl.program_id(0)
    nr = out_ref.shape[0]
    cnt = float(n_total)

    for k in range(nr // tv):
        for r in range(v_per_g):
            blk = resh_ref[k * tv + r * c_dim:k * tv + (r + 1) * c_dim, :]
            xres_scr[pl.Slice(k * tv + r, tv // v_per_g, v_per_g), :] = (
                jnp.transpose(blk))

    s = stats_ref[:, 0:1]
    ss = stats_ref[:, 1:2]
    mean = s / cnt
    var = ss / cnt - mean * mean
    inv = jax.lax.rsqrt(var + 1e-5)
    scale_c = bnw_ref[...] * inv
    shift_c = bnb_ref[...] - mean * scale_c
    scsh = jnp.concatenate([scale_c, shift_c], axis=1)

    oh = oh_ref[...]
    rowsc = jax.lax.dot_general(oh, scsh, (((1,), (0,)), ((), ())),
                                preferred_element_type=jnp.float32)
    scale_r = rowsc[:, 0:1]
    shift_r = rowsc[:, 1:2]

    rows = j * nr + jax.lax.broadcasted_iota(jnp.int32, (nr, 1), 0)
    y = jnp.where(rows < d_rows, a_ref[...], base_ref[...])
    out_ref[...] = jnp.maximum(y * scale_r + shift_r + xres_scr[...], 0.0)


def kernel(x, Wl, bl, Wr, br, att, b, bn_w, bn_b):
    NM, C, T, V = x.shape
    N = NM * T * V
    G = NM * T
    D = 6 * G
    TV = T * V

    xT = jnp.transpose(x, (1, 3, 2, 0))
    bnw2 = bn_w.reshape(C, 1)
    bnb2 = bn_b.reshape(C, 1)

    TS = 8
    p0 = pl.pallas_call(
        functools.partial(_pass0_body, V, TS),
        grid=(T // TS,),
        in_specs=[pl.BlockSpec((C, V, TS, NM), lambda j: (0, 0, j, 0))],
        out_specs=[
            pl.BlockSpec((NM, TS * V, C), lambda j: (0, j, 0)),
            pl.BlockSpec((NM, TS, 6, C), lambda j: (0, j, 0, 0)),
        ],
        out_shape=[
            jax.ShapeDtypeStruct((NM, TV, C), jnp.float32),
            jax.ShapeDtypeStruct((NM, T, 6, C), jnp.float32),
        ],
        compiler_params=pltpu.CompilerParams(
            dimension_semantics=("arbitrary",)),
    )
    resh3, src4 = p0(xT)
    resh = resh3.reshape(N, C)
    srcflat = src4.reshape(D, C)

    GB = 256
    grid1 = G // GB
    nb = V * GB
    na = 6 * GB

    ohb = jnp.asarray(_onehot_np(0, nb, TV, V))
    oha = jnp.asarray(np.concatenate(
        [_onehot_np(jj * na, na, TV, V) for jj in range(7)], axis=0))

    p1 = pl.pallas_call(
        functools.partial(_pass1_body, 3, GB, TV, V, D),
        grid=(grid1,),
        in_specs=[
            pl.BlockSpec((nb, C), lambda j: (j, 0)),
            pl.BlockSpec((na, C), lambda j: (j, 0)),
            pl.BlockSpec((na, C), lambda j: (j, 0)),
            pl.BlockSpec((3, C, C), lambda j: (0, 0, 0)),
            pl.BlockSpec((3, C), lambda j: (0, 0)),
            pl.BlockSpec((3, C, C), lambda j: (0, 0, 0)),
            pl.BlockSpec((3, C), lambda j: (0, 0)),
            pl.BlockSpec((3, C), lambda j: (0, 0)),
            pl.BlockSpec((3, C), lambda j: (0, 0)),
            pl.BlockSpec((nb, C), lambda j: (0, 0)),
            pl.BlockSpec((na, C), lambda j: (jax.lax.rem(j, 7), 0)),
        ],
        out_specs=[
            pl.BlockSpec((nb, C), lambda j: (j, 0)),
            pl.BlockSpec((na, C), lambda j: (j, 0)),
            pl.BlockSpec((C, 2), lambda j: (0, 0)),
        ],
        out_shape=[
            jax.ShapeDtypeStruct((N, C), jnp.float32),
            jax.ShapeDtypeStruct((35 * na, C), jnp.float32),
            jax.ShapeDtypeStruct((C, 2), jnp.float32),
        ],
        scratch_shapes=[
            pltpu.VMEM((na, C), jnp.float32),
            pltpu.VMEM((na, C), jnp.float32),
            pltpu.VMEM((na, C), jnp.float32),
            pltpu.VMEM((na, C), jnp.float32),
        ],
        compiler_params=pltpu.CompilerParams(
            dimension_semantics=("arbitrary",)),
    )
    base, a_out, stats = p1(resh, resh, srcflat, Wl, bl, Wr, br, att, b,
                            ohb, oha)

    R2 = 4 * TV
    grid2 = N // R2
    a_blocks = a_out.shape[0] // R2
    oh2 = jnp.asarray(_onehot_np(0, R2, TV, V))

    p2 = pl.pallas_call(
        functools.partial(_pass2_body, N, D, TV, V, C),
        grid=(grid2,),
        in_specs=[
            pl.BlockSpec((R2, C), lambda j: (j, 0)),
            pl.BlockSpec((R2, C),
                         lambda j: (jnp.minimum(j, a_blocks - 1), 0)),
            pl.BlockSpec((R2, C), lambda j: (j, 0)),
            pl.BlockSpec((C, 2), lambda j: (0, 0)),
            pl.BlockSpec((C, 1), lambda j: (0, 0)),
            pl.BlockSpec((C, 1), lambda j: (0, 0)),
            pl.BlockSpec((R2, C), lambda j: (0, 0)),
        ],
        out_specs=pl.BlockSpec((R2, C), lambda j: (j, 0)),
        out_shape=jax.ShapeDtypeStruct((N, C), jnp.float32),
        scratch_shapes=[pltpu.VMEM((R2, C), jnp.float32)],
        compiler_params=pltpu.CompilerParams(
            dimension_semantics=("arbitrary",)),
    )
    out = p2(base, a_out, resh, stats, bnw2, bnb2, oh2)
    return out.reshape(NM, C, T, V)


# speedup vs baseline: 19.5120x; 2.5716x over previous
import functools

import jax
import jax.numpy as jnp
import numpy as np
from jax.experimental import pallas as pl
from jax.experimental.pallas import tpu as pltpu

_ADJ = (np.array(
    [[1, 0, 0, 0, 1, 0], [1, 1, 0, 0, 0, 1], [1, 0, 1, 0, 0, 1],
     [1, 0, 0, 1, 0, 0], [1, 0, 0, 0, 1, 0], [0, 1, 1, 0, 0, 1]],
    dtype=np.int64).T != 0)

_NBRS = [tuple(c for c in range(6) if _ADJ[r, c]) for r in range(6)]

_SLOPE = 0.2


def _leaky(v):
    return jnp.where(v >= 0, v, _SLOPE * v)


def _matmul_t(a, w):
    return jax.lax.dot_general(a, w, (((1,), (1,)), ((), ())),
                               preferred_element_type=jnp.float32)


def _onehot_np(row0, n_rows, period, group):
    ch = ((row0 + np.arange(n_rows)) % period) // group
    return (ch[:, None] == np.arange(64)[None, :]).astype(np.float32)


def _pass0_body(v_per_g, t_per_step, xt_ref, resh_ref, src_ref):
    for tt in range(t_per_step):
        for v in range(v_per_g):
            tv = jnp.transpose(xt_ref[:, v, tt, :])
            resh_ref[:, tt * v_per_g + v, :] = tv
            if v < 6:
                src_ref[:, tt, v, :] = tv


def _pass1_body(nlayer, gb, tv, v_per_g, d_rows,
                xg_ref, xd_ref, xs_ref, wl_ref, bl_ref, wr_ref, br_ref,
                att_ref, b_ref, ohb_ref, oha_ref,
                base_ref, a_ref, stats_ref,
                s_scr, xld_scr, xrd_scr, a_scr):
    j = pl.program_id(0)
    xg = xg_ref[...]
    xd = xd_ref[...]
    xs = xs_ref[...]

    nr_b = xg.shape[0]
    nr_a = xd.shape[0]
    ng = nr_a // 6

    base_acc = jnp.zeros((nr_b, 64), jnp.float32)
    a_scr[...] = jnp.zeros((nr_a, 64), jnp.float32)

    for i in range(nlayer):
        wl = wl_ref[i]
        wr = wr_ref[i]
        bl_i = bl_ref[i:i + 1, :]
        br_i = br_ref[i:i + 1, :]
        att_i = att_ref[i:i + 1, :]
        b_i = b_ref[i:i + 1, :]

        xl = _matmul_t(xg, wl) + bl_i
        s_scr[...] = _matmul_t(xs, wl) + bl_i
        xld_scr[...] = _matmul_t(xd, wl) + bl_i
        xrd_scr[...] = _matmul_t(xd, wr) + br_i

        for r in range(6):
            dl_r = xld_scr[pl.Slice(r, ng, 6), :]
            dr_r = xrd_scr[pl.Slice(r, ng, 6), :]
            m = jnp.sum(_leaky(dl_r + dr_r) * att_i, axis=1, keepdims=True)
            sl_r = m
            s_list = []
            e_list = []
            for c in _NBRS[r]:
                sc = s_scr[pl.Slice(c, ng, 6), :]
                e = jnp.sum(_leaky(sc + dr_r) * att_i, axis=1, keepdims=True)
                s_list.append(sc)
                e_list.append(e)
                m = jnp.maximum(m, e)
            ex_self = jnp.exp(sl_r - m)
            den = ex_self
            mix = ex_self * dl_r
            for sc, e in zip(s_list, e_list):
                ex = jnp.exp(e - m)
                den = den + ex
                mix = mix + ex * sc
            sl_idx = pl.Slice(r, ng, 6)
            a_scr[sl_idx, :] = (a_scr[sl_idx, :]
                                + mix / (den + 1e-16) + b_i)

        base_acc = base_acc + xl + b_i

    a_acc = a_scr[...]
    base_ref[...] = base_acc
    a_ref[...] = a_acc

    rows_b = (j * nr_b
              + jax.lax.broadcasted_iota(jnp.int32, (nr_b, 1), 0))
    tail = (rows_b >= d_rows).astype(jnp.float32)
    rs_b = jnp.sum(base_acc, axis=1, keepdims=True) * tail
    rss_b = jnp.sum(base_acc * base_acc, axis=1, keepdims=True) * tail
    rs_a = jnp.sum(a_acc, axis=1, keepdims=True)
    rss_a = jnp.sum(a_acc * a_acc, axis=1, keepdims=True)

    oh_b = ohb_ref[...]
    oh_a = oha_ref[...]
    vals_b = jnp.concatenate([rs_b, rss_b], axis=1)
    vals_a = jnp.concatenate([rs_a, rss_a], axis=1)
    contrib = (
        jax.lax.dot_general(oh_b, vals_b, (((0,), (0,)), ((), ())),
                            preferred_element_type=jnp.float32)
        + jax.lax.dot_general(oh_a, vals_a, (((0,), (0,)), ((), ())),
                              preferred_element_type=jnp.float32))

    @pl.when(j == 0)
    def _():
        stats_ref[...] = contrib

    @pl.when(j != 0)
    def _():
        stats_ref[...] = stats_ref[...] + contrib


def _pass2_body(n_total, d_rows, tv, v_per_g, t_per_step, nm_a,
                base_ref, a_ref, xt_ref, stats_ref, bnw_ref, bnb_ref,
                out_ref, scsh_scr):
    j = pl.program_id(0)
    cnt = float(n_total)

    s = stats_ref[:, 0:1]
    ss = stats_ref[:, 1:2]
    mean = s / cnt
    var = ss / cnt - mean * mean
    inv = jax.lax.rsqrt(var + 1e-5)
    scale_c = bnw_ref[...] * inv
    shift_c = bnb_ref[...] - mean * scale_c
    scsh_scr[...] = jnp.concatenate([scale_c, shift_c], axis=1)

    nm_total = out_ref.shape[3]
    nm_iota = jax.lax.broadcasted_iota(jnp.int32, (nm_a, 1), 0)
    for tt in range(t_per_step):
        t_glob = j * t_per_step + tt
        row = scsh_scr[pl.ds(t_glob, 1), :]
        sc = row[:, 0:1]
        sh = row[:, 1:2]
        for v in range(v_per_g):
            node_off = v_per_g * t_glob + v
            base_s = base_ref[:, tt * v_per_g + v, :]
            a_s = a_ref[:, tt * v_per_g + v, :]
            nbound = (d_rows - 1 - node_off) // tv
            mask = nm_iota <= nbound
            y_head = jnp.where(mask, a_s, base_s[:nm_a, :])
            y = jnp.concatenate([y_head, base_s[nm_a:, :]], axis=0)
            z = jnp.transpose(y * sc + sh)
            out_ref[:, v, tt, :] = jnp.maximum(
                z + xt_ref[:, v, tt, :], 0.0)


def kernel(x, Wl, bl, Wr, br, att, b, bn_w, bn_b):
    NM, C, T, V = x.shape
    N = NM * T * V
    G = NM * T
    D = 6 * G
    TV = T * V

    xT = jnp.transpose(x, (1, 3, 2, 0))
    bnw2 = bn_w.reshape(C, 1)
    bnb2 = bn_b.reshape(C, 1)

    TS = 8
    p0 = pl.pallas_call(
        functools.partial(_pass0_body, V, TS),
        grid=(T // TS,),
        in_specs=[pl.BlockSpec((C, V, TS, NM), lambda j: (0, 0, j, 0))],
        out_specs=[
            pl.BlockSpec((NM, TS * V, C), lambda j: (0, j, 0)),
            pl.BlockSpec((NM, TS, 6, C), lambda j: (0, j, 0, 0)),
        ],
        out_shape=[
            jax.ShapeDtypeStruct((NM, TV, C), jnp.float32),
            jax.ShapeDtypeStruct((NM, T, 6, C), jnp.float32),
        ],
        compiler_params=pltpu.CompilerParams(
            dimension_semantics=("arbitrary",)),
    )
    resh3, src4 = p0(xT)
    resh = resh3.reshape(N, C)
    srcflat = src4.reshape(D, C)

    GB = 256
    grid1 = G // GB
    nb = V * GB
    na = 6 * GB

    ohb = jnp.asarray(_onehot_np(0, nb, TV, V))
    oha = jnp.asarray(np.concatenate(
        [_onehot_np(jj * na, na, TV, V) for jj in range(7)], axis=0))

    p1 = pl.pallas_call(
        functools.partial(_pass1_body, 3, GB, TV, V, D),
        grid=(grid1,),
        in_specs=[
            pl.BlockSpec((nb, C), lambda j: (j, 0)),
            pl.BlockSpec((na, C), lambda j: (j, 0)),
            pl.BlockSpec((na, C), lambda j: (j, 0)),
            pl.BlockSpec((3, C, C), lambda j: (0, 0, 0)),
            pl.BlockSpec((3, C), lambda j: (0, 0)),
            pl.BlockSpec((3, C, C), lambda j: (0, 0, 0)),
            pl.BlockSpec((3, C), lambda j: (0, 0)),
            pl.BlockSpec((3, C), lambda j: (0, 0)),
            pl.BlockSpec((3, C), lambda j: (0, 0)),
            pl.BlockSpec((nb, C), lambda j: (0, 0)),
            pl.BlockSpec((na, C), lambda j: (jax.lax.rem(j, 7), 0)),
        ],
        out_specs=[
            pl.BlockSpec((nb, C), lambda j: (j, 0)),
            pl.BlockSpec((na, C), lambda j: (j, 0)),
            pl.BlockSpec((C, 2), lambda j: (0, 0)),
        ],
        out_shape=[
            jax.ShapeDtypeStruct((N, C), jnp.float32),
            jax.ShapeDtypeStruct((35 * na, C), jnp.float32),
            jax.ShapeDtypeStruct((C, 2), jnp.float32),
        ],
        scratch_shapes=[
            pltpu.VMEM((na, C), jnp.float32),
            pltpu.VMEM((na, C), jnp.float32),
            pltpu.VMEM((na, C), jnp.float32),
            pltpu.VMEM((na, C), jnp.float32),
        ],
        compiler_params=pltpu.CompilerParams(
            dimension_semantics=("arbitrary",)),
    )
    base, a_out, stats = p1(resh, resh, srcflat, Wl, bl, Wr, br, att, b,
                            ohb, oha)

    base3 = base.reshape(NM, TV, C)
    nm_a = a_out.shape[0] // TV
    a3 = a_out.reshape(nm_a, TV, C)

    TS2 = 8
    p2 = pl.pallas_call(
        functools.partial(_pass2_body, N, D, TV, V, TS2, nm_a),
        grid=(T // TS2,),
        in_specs=[
            pl.BlockSpec((NM, TS2 * V, C), lambda j: (0, j, 0)),
            pl.BlockSpec((nm_a, TS2 * V, C), lambda j: (0, j, 0)),
            pl.BlockSpec((C, V, TS2, NM), lambda j: (0, 0, j, 0)),
            pl.BlockSpec((C, 2), lambda j: (0, 0)),
            pl.BlockSpec((C, 1), lambda j: (0, 0)),
            pl.BlockSpec((C, 1), lambda j: (0, 0)),
        ],
        out_specs=pl.BlockSpec((C, V, TS2, NM), lambda j: (0, 0, j, 0)),
        out_shape=jax.ShapeDtypeStruct((C, V, T, NM), jnp.float32),
        scratch_shapes=[pltpu.VMEM((C, 2), jnp.float32)],
        compiler_params=pltpu.CompilerParams(
            dimension_semantics=("arbitrary",)),
    )
    out_phys = p2(base3, a3, xT, stats, bnw2, bnb2)
    return jnp.transpose(out_phys, (3, 0, 2, 1))
